# baseline (speedup 1.0000x reference)
import functools

import jax
import jax.numpy as jnp
from jax import lax
from jax.experimental import pallas as pl
from jax.experimental.pallas import tpu as pltpu
from jax.experimental.pallas import tpu_sc as plsc

_NC = 2
_NS = 16
_NW = _NC * _NS

_P = 200
_NT = 16384
_D = 32
_IC = _NT // 128
_UPT = _P * _IC // _NW


@functools.lru_cache(maxsize=None)
def _build():
    mesh = plsc.VectorSubcoreMesh(core_axis_name="c", subcore_axis_name="s")

    @functools.partial(
        pl.kernel,
        mesh=mesh,
        compiler_params=pltpu.CompilerParams(
            use_tc_tiling_on_sc=False, needs_layout_passes=False
        ),
        out_type=jax.ShapeDtypeStruct((_P, _D // 8, _IC, 8, 128), jnp.float32),
        scratch_types=[
            pltpu.VMEM((128,), jnp.int32),
            pltpu.VMEM((128,), jnp.int32),
            pltpu.VMEM((128, _D), jnp.float32),
            pltpu.VMEM((128, _D), jnp.float32),
            pltpu.VMEM((_D, 128), jnp.float32),
            pltpu.VMEM((_D, 128), jnp.float32),
            pltpu.SemaphoreType.DMA,
            pltpu.SemaphoreType.DMA,
            pltpu.SemaphoreType.DMA,
            pltpu.SemaphoreType.DMA,
            pltpu.SemaphoreType.DMA,
            pltpu.SemaphoreType.DMA,
        ],
    )
    def k(x4_hbm, lut_hbm, out_hbm, i0, i1, r0, r1, t0, t1,
          g0, g1, o0, o1, s0, s1):
        idx_v = (i0, i1)
        rows_v = (r0, r1)
        tile_v = (t0, t1)
        gsem = (g0, g1)
        osem = (o0, o1)
        isem = (s0, s1)
        wid = lax.axis_index("s") * _NC + lax.axis_index("c")
        u0 = wid * _UPT

        toks = [
            jnp.arange(16, dtype=jnp.int32) + t8 * 16 for t8 in range(8)
        ]

        def unit_coords(u):
            j = u // _IC
            ic = u % _IC
            return j, ic, j // 8, j % 8

        def idx_copy(u, b):
            _, ic, jr, j8 = unit_coords(u)
            return pltpu.make_async_copy(
                x4_hbm.at[jr, ic, j8], idx_v[b], isem[b]
            )

        def gather(b):
            return pltpu.make_async_copy(
                lut_hbm.at[idx_v[b]], rows_v[b], gsem[b]
            )

        def out_copies(u, b):
            j, ic, _, _ = unit_coords(u)
            return [
                pltpu.make_async_copy(
                    tile_v[b].at[pl.ds(fr * 8, 8)],
                    out_hbm.at[j, fr, ic],
                    osem[b],
                )
                for fr in range(4)
            ]

        idx_copy(u0, 0).start()
        idx_copy(u0, 0).wait()
        gather(0).start()
        idx_copy(u0 + 1, 1).start()

        @pl.loop(0, _UPT, step=2)
        def _outer(n0):
            for b in range(2):
                n = n0 + b
                u = u0 + n

                gather(b).wait()

                @pl.when(n + 1 < _UPT)
                def _():
                    idx_copy(u + 1, 1 - b).wait()
                    gather(1 - b).start()

                @pl.when(n + 2 < _UPT)
                def _():
                    idx_copy(u + 2, b).start()

                @pl.when(n >= 2)
                def _():
                    for c in out_copies(u - 2, b):
                        c.wait()

                for f in range(_D):
                    fv = jnp.full((16,), f, dtype=jnp.int32)
                    for t8 in range(8):
                        vals = plsc.load_gather(rows_v[b], [toks[t8], fv])
                        tile_v[b][f, pl.ds(t8 * 16, 16)] = vals

                for c in out_copies(u, b):
                    c.start()

        for n in (_UPT - 2, _UPT - 1):
            for c in out_copies(u0 + n, n % 2):
                c.wait()

    return k


def kernel(x, lut):
    x4 = x.T.reshape(_P // 8, 8, _IC, 128).transpose(0, 2, 1, 3)
    out5 = _build()(x4, lut)
    out = out5.transpose(2, 4, 0, 1, 3).reshape(_NT, _P, _D)
    return out


# speedup vs baseline: 1.7730x; 1.7730x over previous
import functools

import jax
import jax.numpy as jnp
from jax import lax
from jax.experimental import pallas as pl
from jax.experimental.pallas import tpu as pltpu
from jax.experimental.pallas import tpu_sc as plsc

_NC = 2
_NS = 16
_NW = _NC * _NS

_P = 200
_NT = 16384
_D = 32
_IC = _NT // 128
_UPT = _P * _IC // _NW


@functools.lru_cache(maxsize=None)
def _build():
    mesh = plsc.VectorSubcoreMesh(core_axis_name="c", subcore_axis_name="s")

    @functools.partial(
        pl.kernel,
        mesh=mesh,
        compiler_params=pltpu.CompilerParams(
            use_tc_tiling_on_sc=False, needs_layout_passes=False
        ),
        out_type=jax.ShapeDtypeStruct((_P, _D // 8, _IC, 8, 128), jnp.float32),
        scratch_types=[
            pltpu.VMEM((128,), jnp.int32),
            pltpu.VMEM((128,), jnp.int32),
            pltpu.VMEM((128, _D), jnp.float32),
            pltpu.VMEM((128, _D), jnp.float32),
            pltpu.VMEM((_D, 128), jnp.float32),
            pltpu.VMEM((_D, 128), jnp.float32),
            pltpu.SemaphoreType.DMA,
            pltpu.SemaphoreType.DMA,
            pltpu.SemaphoreType.DMA,
            pltpu.SemaphoreType.DMA,
            pltpu.SemaphoreType.DMA,
            pltpu.SemaphoreType.DMA,
        ],
    )
    def k(x4_hbm, lut_hbm, out_hbm, i0, i1, r0, r1, t0, t1,
          g0, g1, o0, o1, s0, s1):
        idx_v = (i0, i1)
        rows_v = (r0, r1)
        tile_v = (t0, t1)
        gsem = (g0, g1)
        osem = (o0, o1)
        isem = (s0, s1)
        wid = lax.axis_index("s") * _NC + lax.axis_index("c")
        u0 = wid * _UPT

        toks = [
            jnp.arange(16, dtype=jnp.int32) + t8 * 16 for t8 in range(8)
        ]

        def unit_coords(u):
            j = u // _IC
            ic = u % _IC
            return j, ic, j // 8, j % 8

        def idx_copy(u, b):
            _, ic, jr, j8 = unit_coords(u)
            return pltpu.make_async_copy(
                x4_hbm.at[jr, ic, j8], idx_v[b], isem[b]
            )

        def gather(b):
            return pltpu.make_async_copy(
                lut_hbm.at[idx_v[b]], rows_v[b], gsem[b]
            )

        def out_copies(u, b):
            j, ic, _, _ = unit_coords(u)
            return [
                pltpu.make_async_copy(
                    tile_v[b].at[pl.ds(fr * 8, 8)],
                    out_hbm.at[j, fr, ic],
                    osem[b],
                )
                for fr in range(4)
            ]

        idx_copy(u0, 0).start()
        idx_copy(u0, 0).wait()
        gather(0).start()
        idx_copy(u0 + 1, 1).start()

        @pl.loop(0, _UPT, step=2)
        def _outer(n0):
            for b in range(2):
                n = n0 + b
                u = u0 + n

                gather(b).wait()

                @pl.when(n + 1 < _UPT)
                def _():
                    idx_copy(u + 1, 1 - b).wait()
                    gather(1 - b).start()

                @pl.when(n + 2 < _UPT)
                def _():
                    idx_copy(u + 2, b).start()

                @pl.when(n >= 2)
                def _():
                    for c in out_copies(u - 2, b):
                        c.wait()

                @plsc.parallel_loop(0, _D, unroll=8)
                def _row(f):
                    fv = jnp.full((16,), f, dtype=jnp.int32)
                    for t8 in range(8):
                        vals = plsc.load_gather(rows_v[b], [toks[t8], fv])
                        tile_v[b][f, pl.ds(t8 * 16, 16)] = vals

                for c in out_copies(u, b):
                    c.start()

        for n in (_UPT - 2, _UPT - 1):
            for c in out_copies(u0 + n, n % 2):
                c.wait()

    return k


def kernel(x, lut):
    x4 = x.T.reshape(_P // 8, 8, _IC, 128).transpose(0, 2, 1, 3)
    out5 = _build()(x4, lut)
    out = out5.transpose(2, 4, 0, 1, 3).reshape(_NT, _P, _D)
    return out


# speedup vs baseline: 2.7489x; 1.5504x over previous
import functools

import jax
import jax.numpy as jnp
from jax import lax
from jax.experimental import pallas as pl
from jax.experimental.pallas import tpu as pltpu
from jax.experimental.pallas import tpu_sc as plsc

_NC = 2
_NS = 16
_NW = _NC * _NS

_P = 200
_NT = 16384
_D = 32
_IC = _NT // 128
_UPT = _P * _IC // _NW


@functools.lru_cache(maxsize=None)
def _build():
    mesh = plsc.VectorSubcoreMesh(core_axis_name="c", subcore_axis_name="s")

    @functools.partial(
        pl.kernel,
        mesh=mesh,
        compiler_params=pltpu.CompilerParams(
            use_tc_tiling_on_sc=False, needs_layout_passes=False
        ),
        out_type=jax.ShapeDtypeStruct((_P, _D // 8, _IC, 8, 128), jnp.float32),
        scratch_types=[
            pltpu.VMEM((128,), jnp.int32),
            pltpu.VMEM((128,), jnp.int32),
            pltpu.VMEM((128, _D), jnp.float32),
            pltpu.VMEM((128, _D), jnp.float32),
            pltpu.VMEM((_D, 128), jnp.float32),
            pltpu.VMEM((_D, 128), jnp.float32),
            pltpu.SemaphoreType.DMA,
            pltpu.SemaphoreType.DMA,
            pltpu.SemaphoreType.DMA,
            pltpu.SemaphoreType.DMA,
            pltpu.SemaphoreType.DMA,
            pltpu.SemaphoreType.DMA,
        ],
    )
    def k(x4_hbm, lut_hbm, out_hbm, i0, i1, r0, r1, t0, t1,
          g0, g1, o0, o1, s0, s1):
        idx_v = (i0, i1)
        rows_v = (r0, r1)
        tile_v = (t0, t1)
        gsem = (g0, g1)
        osem = (o0, o1)
        isem = (s0, s1)
        wid = lax.axis_index("s") * _NC + lax.axis_index("c")
        u0 = wid * _UPT

        iot = jnp.arange(16, dtype=jnp.int32)

        def unit_coords(u):
            j = u // _IC
            ic = u % _IC
            return j, ic, j // 8, j % 8

        def idx_copy(u, b):
            _, ic, jr, j8 = unit_coords(u)
            return pltpu.make_async_copy(
                x4_hbm.at[jr, ic, j8], idx_v[b], isem[b]
            )

        def gather(b):
            return pltpu.make_async_copy(
                lut_hbm.at[idx_v[b]], rows_v[b], gsem[b]
            )

        def out_copies(u, b):
            j, ic, _, _ = unit_coords(u)
            return [
                pltpu.make_async_copy(
                    tile_v[b].at[pl.ds(fr * 8, 8)],
                    out_hbm.at[j, fr, ic],
                    osem[b],
                )
                for fr in range(4)
            ]

        idx_copy(u0, 0).start()
        idx_copy(u0, 0).wait()
        gather(0).start()
        idx_copy(u0 + 1, 1).start()

        @pl.loop(0, _UPT, step=2)
        def _outer(n0):
            for b in range(2):
                n = n0 + b
                u = u0 + n

                gather(b).wait()

                @pl.when(n + 1 < _UPT)
                def _():
                    idx_copy(u + 1, 1 - b).wait()
                    gather(1 - b).start()

                @pl.when(n + 2 < _UPT)
                def _():
                    idx_copy(u + 2, b).start()

                @pl.when(n >= 2)
                def _():
                    for c in out_copies(u - 2, b):
                        c.wait()

                @plsc.parallel_loop(0, 16, unroll=8)
                def _diag(d):
                    pd = lax.bitwise_and(iot + d, 15)
                    fvs = [iot, iot + 16]
                    for t0 in range(0, 128, 16):
                        tv = pd + t0
                        for fi in range(_D // 16):
                            vals = plsc.load_gather(
                                rows_v[b], [tv, fvs[fi]]
                            )
                            plsc.store_scatter(
                                tile_v[b], [fvs[fi], tv], vals
                            )

                for c in out_copies(u, b):
                    c.start()

        for n in (_UPT - 2, _UPT - 1):
            for c in out_copies(u0 + n, n % 2):
                c.wait()

    return k


def kernel(x, lut):
    x4 = x.T.reshape(_P // 8, 8, _IC, 128).transpose(0, 2, 1, 3)
    out5 = _build()(x4, lut)
    out = out5.transpose(2, 4, 0, 1, 3).reshape(_NT, _P, _D)
    return out


# speedup vs baseline: 3.1376x; 1.1414x over previous
import functools

import jax
import jax.numpy as jnp
from jax import lax
from jax.experimental import pallas as pl
from jax.experimental.pallas import tpu as pltpu
from jax.experimental.pallas import tpu_sc as plsc

_NC = 2
_NS = 16
_NW = _NC * _NS

_P = 200
_NT = 16384
_D = 32
_IC = _NT // 128
_G = 4
_ICG = _IC // _G
_UPT = _P * _ICG // _NW


@functools.lru_cache(maxsize=None)
def _build():
    mesh = plsc.VectorSubcoreMesh(core_axis_name="c", subcore_axis_name="s")

    @functools.partial(
        pl.kernel,
        mesh=mesh,
        compiler_params=pltpu.CompilerParams(
            use_tc_tiling_on_sc=False, needs_layout_passes=False
        ),
        out_type=jax.ShapeDtypeStruct((_P, _D // 8, _IC * 1024), jnp.float32),
        scratch_types=[
            pltpu.VMEM((_G * 128,), jnp.int32),
            pltpu.VMEM((_G * 128,), jnp.int32),
            pltpu.VMEM((_G * 128, _D), jnp.float32),
            pltpu.VMEM((_G * 128, _D), jnp.float32),
            pltpu.VMEM((_G * _D * 128,), jnp.float32),
            pltpu.VMEM((_G * _D * 128,), jnp.float32),
            pltpu.SemaphoreType.DMA,
            pltpu.SemaphoreType.DMA,
            pltpu.SemaphoreType.DMA,
            pltpu.SemaphoreType.DMA,
            pltpu.SemaphoreType.DMA,
            pltpu.SemaphoreType.DMA,
        ],
    )
    def k(x4_hbm, lut_hbm, out_hbm, i0, i1, r0, r1, t0, t1,
          g0, g1, o0, o1, s0, s1):
        idx_v = (i0, i1)
        rows_v = (r0, r1)
        tile_v = (t0, t1)
        gsem = (g0, g1)
        osem = (o0, o1)
        isem = (s0, s1)
        wid = lax.axis_index("s") * _NC + lax.axis_index("c")
        u0 = wid * _UPT

        iot = jnp.arange(16, dtype=jnp.int32)
        fvs = [iot, iot + 16]
        rc128 = [
            [
                ((f0 + iot) // 8) * 4096 + ((f0 + iot) % 8) * 128
                + ic2 * 1024
                for ic2 in range(_G)
            ]
            for f0 in (0, 16)
        ]

        def unit_coords(u):
            j = u // _ICG
            icg = u % _ICG
            return j, icg, j // 8, j % 8

        def idx_copies(u, b):
            _, icg, jr, j8 = unit_coords(u)
            return [
                pltpu.make_async_copy(
                    x4_hbm.at[jr, icg * _G + g, j8],
                    idx_v[b].at[pl.ds(g * 128, 128)],
                    isem[b],
                )
                for g in range(_G)
            ]

        def gather(b):
            return pltpu.make_async_copy(
                lut_hbm.at[idx_v[b]], rows_v[b], gsem[b]
            )

        def out_copies(u, b):
            j, icg, _, _ = unit_coords(u)
            return [
                pltpu.make_async_copy(
                    tile_v[b].at[pl.ds(fr * _G * 1024, _G * 1024)],
                    out_hbm.at[j, fr, pl.ds(icg * _G * 1024, _G * 1024)],
                    osem[b],
                )
                for fr in range(4)
            ]

        for c in idx_copies(u0, 0):
            c.start()
        for c in idx_copies(u0, 0):
            c.wait()
        gather(0).start()
        for c in idx_copies(u0 + 1, 1):
            c.start()

        @pl.loop(0, _UPT, step=2)
        def _outer(n0):
            for b in range(2):
                n = n0 + b
                u = u0 + n

                gather(b).wait()

                @pl.when(n + 1 < _UPT)
                def _():
                    for c in idx_copies(u + 1, 1 - b):
                        c.wait()
                    gather(1 - b).start()

                @pl.when(n + 2 < _UPT)
                def _():
                    for c in idx_copies(u + 2, b):
                        c.start()

                @pl.when(n >= 2)
                def _():
                    for c in out_copies(u - 2, b):
                        c.wait()

                @plsc.parallel_loop(0, 16, unroll=2)
                def _diag(d):
                    pd = lax.bitwise_and(iot + d, 15)
                    for ic2 in range(_G):
                        for tb in range(8):
                            tv = pd + (ic2 * 128 + tb * 16)
                            il = pd + tb * 16
                            for fi in range(_D // 16):
                                vals = plsc.load_gather(
                                    rows_v[b], [tv, fvs[fi]]
                                )
                                plsc.store_scatter(
                                    tile_v[b],
                                    [rc128[fi][ic2] + il],
                                    vals,
                                )

                for c in out_copies(u, b):
                    c.start()

        for n in (_UPT - 2, _UPT - 1):
            for c in out_copies(u0 + n, n % 2):
                c.wait()

    return k


def kernel(x, lut):
    x4 = x.T.reshape(_P // 8, 8, _IC, 128).transpose(0, 2, 1, 3)
    out3 = _build()(x4, lut)
    out5 = out3.reshape(_P, _D // 8, _IC, 8, 128)
    out = out5.transpose(2, 4, 0, 1, 3).reshape(_NT, _P, _D)
    return out


# speedup vs baseline: 3.1982x; 1.0193x over previous
import functools

import jax
import jax.numpy as jnp
from jax import lax
from jax.experimental import pallas as pl
from jax.experimental.pallas import tpu as pltpu
from jax.experimental.pallas import tpu_sc as plsc

_NC = 2
_NS = 16
_NW = _NC * _NS

_P = 200
_NT = 16384
_D = 32
_IC = _NT // 128
_G = 4
_ICG = _IC // _G
_UPT = _P * _ICG // _NW


@functools.lru_cache(maxsize=None)
def _build():
    mesh = plsc.VectorSubcoreMesh(core_axis_name="c", subcore_axis_name="s")

    @functools.partial(
        pl.kernel,
        mesh=mesh,
        compiler_params=pltpu.CompilerParams(
            use_tc_tiling_on_sc=False, needs_layout_passes=False
        ),
        out_type=jax.ShapeDtypeStruct((_P, _D // 8, _IC * 1024), jnp.float32),
        scratch_types=[
            pltpu.VMEM((_G * 128,), jnp.int32),
            pltpu.VMEM((_G * 128,), jnp.int32),
            pltpu.VMEM((_G * 128, _D), jnp.float32),
            pltpu.VMEM((_G * 128, _D), jnp.float32),
            pltpu.VMEM((_G * _D * 128,), jnp.float32),
            pltpu.VMEM((_G * _D * 128,), jnp.float32),
            pltpu.SemaphoreType.DMA,
            pltpu.SemaphoreType.DMA,
            pltpu.SemaphoreType.DMA,
            pltpu.SemaphoreType.DMA,
            pltpu.SemaphoreType.DMA,
            pltpu.SemaphoreType.DMA,
        ],
    )
    def k(x4_hbm, lut_hbm, out_hbm, i0, i1, r0, r1, t0, t1,
          g0, g1, o0, o1, s0, s1):
        idx_v = (i0, i1)
        rows_v = (r0, r1)
        tile_v = (t0, t1)
        gsem = (g0, g1)
        osem = (o0, o1)
        isem = (s0, s1)
        wid = lax.axis_index("s") * _NC + lax.axis_index("c")
        u0 = wid * _UPT

        iot = jnp.arange(16, dtype=jnp.int32)
        fvs = [iot, iot + 16]
        rc128 = [
            [
                ((f0 + iot) // 8) * 4096 + ((f0 + iot) % 8) * 128
                + ic2 * 1024
                for ic2 in range(_G)
            ]
            for f0 in (0, 16)
        ]

        def unit_coords(u):
            j = u // _ICG
            icg = u % _ICG
            return j, icg, j // 8, j % 8

        def idx_copies(u, b):
            _, icg, jr, j8 = unit_coords(u)
            return [
                pltpu.make_async_copy(
                    x4_hbm.at[jr, icg * _G + g, j8],
                    idx_v[b].at[pl.ds(g * 128, 128)],
                    isem[b],
                )
                for g in range(_G)
            ]

        def gather(b):
            return pltpu.make_async_copy(
                lut_hbm.at[idx_v[b]], rows_v[b], gsem[b]
            )

        def out_copies(u, b):
            j, icg, _, _ = unit_coords(u)
            return [
                pltpu.make_async_copy(
                    tile_v[b].at[pl.ds(fr * _G * 1024, _G * 1024)],
                    out_hbm.at[j, fr, pl.ds(icg * _G * 1024, _G * 1024)],
                    osem[b],
                )
                for fr in range(4)
            ]

        def scale_idx(b):
            @plsc.parallel_loop(0, _G * 128, step=16, unroll=4)
            def _(i):
                idx_v[b][pl.ds(i, 16)] = idx_v[b][pl.ds(i, 16)] * 4

        for c in idx_copies(u0, 0):
            c.start()
        for c in idx_copies(u0, 0):
            c.wait()
        scale_idx(0)
        gather(0).start()
        for c in idx_copies(u0 + 1, 1):
            c.start()

        @pl.loop(0, _UPT, step=2)
        def _outer(n0):
            for b in range(2):
                n = n0 + b
                u = u0 + n

                gather(b).wait()

                @pl.when(n + 1 < _UPT)
                def _():
                    for c in idx_copies(u + 1, 1 - b):
                        c.wait()
                    scale_idx(1 - b)
                    gather(1 - b).start()

                @pl.when(n + 2 < _UPT)
                def _():
                    for c in idx_copies(u + 2, b):
                        c.start()

                @pl.when(n >= 2)
                def _():
                    for c in out_copies(u - 2, b):
                        c.wait()

                @plsc.parallel_loop(0, 16, unroll=2)
                def _diag(d):
                    pd = lax.bitwise_and(iot + d, 15)
                    for ic2 in range(_G):
                        for tb in range(8):
                            tv = pd + (ic2 * 128 + tb * 16)
                            il = pd + tb * 16
                            for fi in range(_D // 16):
                                vals = plsc.load_gather(
                                    rows_v[b], [tv, fvs[fi]]
                                )
                                plsc.store_scatter(
                                    tile_v[b],
                                    [rc128[fi][ic2] + il],
                                    vals,
                                )

                for c in out_copies(u, b):
                    c.start()

        for n in (_UPT - 2, _UPT - 1):
            for c in out_copies(u0 + n, n % 2):
                c.wait()

    return k


def kernel(x, lut):
    x4 = x.T.reshape(_P // 8, 8, _IC, 128).transpose(0, 2, 1, 3)
    lutv = jnp.pad(lut, ((0, 0), (0, 96))).reshape(4 * lut.shape[0], _D)
    out3 = _build()(x4, lutv)
    out5 = out3.reshape(_P, _D // 8, _IC, 8, 128)
    out = out5.transpose(2, 4, 0, 1, 3).reshape(_NT, _P, _D)
    return out
